# Initial kernel scaffold; baseline (speedup 1.0000x reference)
import jax
import jax.numpy as jnp
from jax.experimental import pallas as pl


def kernel(x, emb, W1, b1, W2, b2, W3, b3):
    raise NotImplementedError("write your pallas kernel here")


# speedup vs baseline: 12.0776x; 12.0776x over previous
import functools

import jax
import jax.numpy as jnp
from jax import lax
from jax.experimental import pallas as pl
from jax.experimental.pallas import tpu as pltpu
from jax.experimental.pallas import tpu_sc as plsc


def _sc_pool(x3, emb, nc, ns):
    nw = nc * ns
    _, rpw, seq = x3.shape
    _, e = emb.shape
    nlane = 16
    nv = e // nlane
    inv_l = jnp.float32(1.0 / seq)

    mesh = plsc.VectorSubcoreMesh(core_axis_name="c", subcore_axis_name="s")
    chunk = 64
    nchunk = rpw // chunk

    @functools.partial(
        pl.kernel,
        mesh=mesh,
        out_type=jax.ShapeDtypeStruct((nw * rpw, e), jnp.float32),
        scratch_types=[
            pltpu.VMEM((rpw, seq), jnp.int32),
            pltpu.VMEM((seq, e), jnp.float32),
            pltpu.VMEM((seq, e), jnp.float32),
            pltpu.VMEM((chunk, e), jnp.float32),
            pltpu.VMEM((chunk, e), jnp.float32),
            pltpu.SemaphoreType.DMA,
            pltpu.SemaphoreType.DMA,
            pltpu.SemaphoreType.DMA,
            pltpu.SemaphoreType.DMA,
        ],
    )
    def body(x_hbm, emb_hbm, out_hbm, idx_v, rows0_v, rows1_v,
             pool0_v, pool1_v, gsem0, gsem1, osem0, osem1):
        wid = lax.axis_index("s") * nc + lax.axis_index("c")
        base = wid * rpw
        pltpu.sync_copy(x_hbm.at[wid], idx_v)

        def start(r, buf, sem):
            pltpu.make_async_copy(emb_hbm.at[idx_v.at[r]], buf, sem).start()

        def wait(buf, sem):
            pltpu.make_async_copy(emb_hbm.at[idx_v.at[0]], buf, sem).wait()

        def accum(buf, pool_buf, lr):
            def inner(j, acc):
                return tuple(acc[k] + buf[j, pl.ds(nlane * k, nlane)]
                             for k in range(nv))

            acc = lax.fori_loop(
                0, seq, inner,
                tuple(jnp.zeros((nlane,), jnp.float32) for _ in range(nv)))
            for k in range(nv):
                pool_buf[lr, pl.ds(nlane * k, nlane)] = acc[k] * inv_l

        start(0, rows0_v, gsem0)
        start(1, rows1_v, gsem1)
        pool_bufs = (pool0_v, pool1_v)
        out_sems = (osem0, osem1)

        def outer(c2, _):
            for cc in range(2):
                c = 2 * c2 + cc
                pool_buf, osem = pool_bufs[cc], out_sems[cc]
                out_slc = out_hbm.at[pl.ds(base + c * chunk, chunk)]

                @pl.when(c2 > 0)
                def _():
                    pltpu.make_async_copy(pool_buf, out_slc, osem).wait()

                def pair(p, _):
                    r0 = c * chunk + 2 * p
                    wait(rows0_v, gsem0)

                    @pl.when(r0 + 2 < rpw)
                    def _():
                        start(r0 + 2, rows0_v, gsem0)

                    accum(rows0_v, pool_buf, 2 * p)
                    wait(rows1_v, gsem1)

                    @pl.when(r0 + 3 < rpw)
                    def _():
                        start(r0 + 3, rows1_v, gsem1)

                    accum(rows1_v, pool_buf, 2 * p + 1)
                    return 0

                lax.fori_loop(0, chunk // 2, pair, 0)
                pltpu.make_async_copy(pool_buf, out_slc, osem).start()
            return 0

        lax.fori_loop(0, nchunk // 2, outer, 0)
        for cc in range(2):
            c = nchunk - 2 + cc
            pltpu.make_async_copy(
                pool_bufs[cc],
                out_hbm.at[pl.ds(base + c * chunk, chunk)],
                out_sems[cc]).wait()

    return body(x3, emb)


def _tc_mlp(pooled, w1, b1, w2, b2, w3, b3, bt):
    b, e = pooled.shape
    h1 = w1.shape[1]
    h2 = w2.shape[1]
    out = w3.shape[1]

    def body(p_ref, w1_ref, b1_ref, w2_ref, b2_ref, w3_ref, b3_ref, o_ref):
        h = jnp.dot(p_ref[...], w1_ref[...], preferred_element_type=jnp.float32)
        h = jnp.maximum(h + b1_ref[...], 0.0)
        h = jnp.dot(h, w2_ref[...], preferred_element_type=jnp.float32)
        h = jnp.maximum(h + b2_ref[...], 0.0)
        h = jnp.dot(h, w3_ref[...], preferred_element_type=jnp.float32)
        o_ref[...] = h + b3_ref[...]

    zero = lambda i: (0, 0)
    return pl.pallas_call(
        body,
        grid=(b // bt,),
        in_specs=[
            pl.BlockSpec((bt, e), lambda i: (i, 0)),
            pl.BlockSpec((e, h1), zero),
            pl.BlockSpec((1, h1), zero),
            pl.BlockSpec((h1, h2), zero),
            pl.BlockSpec((1, h2), zero),
            pl.BlockSpec((h2, out), zero),
            pl.BlockSpec((1, out), zero),
        ],
        out_specs=pl.BlockSpec((bt, out), lambda i: (i, 0)),
        out_shape=jax.ShapeDtypeStruct((b, out), jnp.float32),
    )(pooled, w1, b1.reshape(1, -1), w2, b2.reshape(1, -1),
      w3, b3.reshape(1, -1))


def kernel(x, emb, W1, b1, W2, b2, W3, b3):
    b, seq = x.shape
    nc, ns = 2, 16
    nw = nc * ns
    rpw = b // nw
    x3 = x.astype(jnp.int32).reshape(nw, rpw, seq)
    pooled = _sc_pool(x3, emb, nc, ns)
    return _tc_mlp(pooled, W1, b1, W2, b2, W3, b3, bt=1024)
